# Initial kernel scaffold; baseline (speedup 1.0000x reference)
import jax
import jax.numpy as jnp
from jax.experimental import pallas as pl


def kernel(x, edge_index, Wl0, bl0, Wr0, Wl1, bl1, Wr1, Wl2, bl2, Wr2, Wl3, bl3, Wr3):
    raise NotImplementedError("write your pallas kernel here")


# speedup vs baseline: 7.4140x; 7.4140x over previous
import functools

import jax
import jax.numpy as jnp
from jax import lax
from jax.experimental import pallas as pl
from jax.experimental.pallas import tpu as pltpu
from jax.experimental.pallas import tpu_sc as plsc

_N = 10000
_E = 320000
_NP = 10240
_CHUNK = 80
_NWORK = 32
_EPW = _E // _NWORK
_NCH = _EPW // _CHUNK
_ROWS_PT = _NP // 16


def _make_agg(d, with_counts):
  mesh = plsc.VectorSubcoreMesh(core_axis_name="c", subcore_axis_name="s")
  out_type = [jax.ShapeDtypeStruct((2, _NP, d), jnp.float32)]
  scratch = [
      pltpu.VMEM((_NCH, _CHUNK), jnp.int32),
      pltpu.VMEM((_NCH, _CHUNK), jnp.int32),
      pltpu.VMEM((_CHUNK, d), jnp.float32),
      pltpu.VMEM_SHARED((_NP, d), jnp.float32),
      pltpu.SemaphoreType.DMA,
  ]
  if with_counts:
    out_type.append(jax.ShapeDtypeStruct((2 * _NP,), jnp.float32))
    scratch += [
        pltpu.VMEM((_CHUNK,), jnp.float32),
        pltpu.VMEM((_ROWS_PT,), jnp.float32),
        pltpu.VMEM_SHARED((_NP,), jnp.float32),
    ]

  def body(x_hbm, src_hbm, dst_hbm, out_hbm, *rest):
    if with_counts:
      cnt_hbm, src_v, dst_v, rows_v, acc, sem, ones_v, zcnt_v, cnt_acc = rest
    else:
      src_v, dst_v, rows_v, acc, sem = rest
    cid = lax.axis_index("c")
    sid = lax.axis_index("s")
    wid = sid * 2 + cid
    tid = sid

    pltpu.sync_copy(src_hbm.at[wid], src_v)
    pltpu.sync_copy(dst_hbm.at[wid], dst_v)

    def zrow(r, c):
      for cc in range(d // 16):
        rows_v[r, pl.ds(cc * 16, 16)] = jnp.zeros((16,), jnp.float32)
      return c
    lax.fori_loop(0, _CHUNK, zrow, 0)
    for j in range(_ROWS_PT // _CHUNK):
      pltpu.sync_copy(
          rows_v, acc.at[pl.ds(tid * _ROWS_PT + j * _CHUNK, _CHUNK)])
    if with_counts:
      def zc(i, c):
        zcnt_v[pl.ds(i * 16, 16)] = jnp.zeros((16,), jnp.float32)
        return c
      lax.fori_loop(0, _ROWS_PT // 16, zc, 0)
      pltpu.sync_copy(zcnt_v, cnt_acc.at[pl.ds(tid * _ROWS_PT, _ROWS_PT)])
      for i in range(_CHUNK // 16):
        ones_v[pl.ds(i * 16, 16)] = jnp.ones((16,), jnp.float32)
    plsc.subcore_barrier()

    def step(k, c):
      pltpu.async_copy(x_hbm.at[src_v.at[k]], rows_v, sem).wait()
      pltpu.sync_copy(rows_v, acc.at[dst_v.at[k]], add=True)
      if with_counts:
        pltpu.sync_copy(ones_v, cnt_acc.at[dst_v.at[k]], add=True)
      return c
    lax.fori_loop(0, _NCH, step, 0)

    plsc.subcore_barrier()
    row0 = tid * _ROWS_PT
    pltpu.sync_copy(acc.at[pl.ds(row0, _ROWS_PT)],
                    out_hbm.at[cid, pl.ds(row0, _ROWS_PT)])
    if with_counts:
      pltpu.sync_copy(cnt_acc.at[pl.ds(row0, _ROWS_PT)],
                      cnt_hbm.at[pl.ds(cid * _NP + row0, _ROWS_PT)])

  return pl.kernel(body, out_type=tuple(out_type), mesh=mesh,
                   scratch_types=tuple(scratch))


_BN = 2048


def _inv_of(cnt_blk):
  c = cnt_blk[0] + cnt_blk[1]
  return (1.0 / jnp.maximum(c, 1.0))[:, None]


def _dot(a, b):
  return jax.lax.dot_general(a, b, (((1,), (0,)), ((), ())),
                             preferred_element_type=jnp.float32)


def _tc_call(body, out_shapes, in_specs, out_specs):
  return pl.pallas_call(
      body,
      grid=(_NP // _BN,),
      in_specs=in_specs,
      out_specs=out_specs,
      out_shape=out_shapes,
  )


def _spec_rows(d):
  return pl.BlockSpec((_BN, d), lambda i: (i, 0))


def _spec_pair(d):
  return pl.BlockSpec((2, _BN, d), lambda i: (0, i, 0))


def _spec_cnt():
  return pl.BlockSpec((2, _BN), lambda i: (0, i))


def _spec_full(r, c):
  return pl.BlockSpec((r, c), lambda i: (0, 0))


def _stage_a(p0, cnt, x, wl0t, bl0, wr0t, wl1t):
  def body(p_ref, c_ref, x_ref, wl_ref, b_ref, wr_ref, w1_ref, h0_ref, t1_ref):
    m = (p_ref[0] + p_ref[1]) * _inv_of(c_ref)
    h0 = jnp.maximum(
        _dot(m, wl_ref[...]) + b_ref[...] + _dot(x_ref[...], wr_ref[...]), 0.0)
    h0_ref[...] = h0
    t1_ref[...] = _dot(h0, w1_ref[...])
  return _tc_call(
      body,
      (jax.ShapeDtypeStruct((_NP, 128), jnp.float32),
       jax.ShapeDtypeStruct((_NP, 128), jnp.float32)),
      [_spec_pair(128), _spec_cnt(), _spec_rows(128), _spec_full(128, 128),
       _spec_full(1, 128), _spec_full(128, 128), _spec_full(128, 128)],
      (_spec_rows(128), _spec_rows(128)),
  )(p0, cnt, x, wl0t, bl0, wr0t, wl1t)


def _stage_b(p1, cnt, h0, bl1, wr1t):
  def body(p_ref, c_ref, h_ref, b_ref, wr_ref, o_ref):
    m = (p_ref[0] + p_ref[1]) * _inv_of(c_ref)
    o_ref[...] = m + b_ref[...] + _dot(h_ref[...], wr_ref[...])
  return _tc_call(
      body,
      jax.ShapeDtypeStruct((_NP, 128), jnp.float32),
      [_spec_pair(128), _spec_cnt(), _spec_rows(128), _spec_full(1, 128),
       _spec_full(128, 128)],
      _spec_rows(128),
  )(p1, cnt, h0, bl1, wr1t)


def _stage_c(p2, cnt, h1, wl2t, bl2, wr2t):
  def body(p_ref, c_ref, h_ref, wl_ref, b_ref, wr_ref, o_ref):
    m = (p_ref[0] + p_ref[1]) * _inv_of(c_ref)
    o_ref[...] = jnp.maximum(
        _dot(m, wl_ref[...]) + b_ref[...] + _dot(h_ref[...], wr_ref[...]), 0.0)
  return _tc_call(
      body,
      jax.ShapeDtypeStruct((_NP, 128), jnp.float32),
      [_spec_pair(128), _spec_cnt(), _spec_rows(128), _spec_full(128, 128),
       _spec_full(1, 128), _spec_full(128, 128)],
      _spec_rows(128),
  )(p2, cnt, h1, wl2t, bl2, wr2t)


def _stage_d(p3, cnt, h2, wl3t, bl3, wr3t):
  def body(p_ref, c_ref, h_ref, wl_ref, b_ref, wr_ref, o_ref):
    m = (p_ref[0] + p_ref[1]) * _inv_of(c_ref)
    o_ref[...] = (_dot(m, wl_ref[...]) + b_ref[...]
                  + _dot(h_ref[...], wr_ref[...]))
  return _tc_call(
      body,
      jax.ShapeDtypeStruct((_NP, 128), jnp.float32),
      [_spec_pair(128), _spec_cnt(), _spec_rows(128), _spec_full(128, 128),
       _spec_full(1, 128), _spec_full(128, 128)],
      _spec_rows(128),
  )(p3, cnt, h2, wl3t, bl3, wr3t)


def _pad_cols(w, n):
  return jnp.concatenate([w, jnp.zeros((w.shape[0], n), jnp.float32)], axis=1)


def _pad_rows(w, n):
  return jnp.concatenate([w, jnp.zeros((n, w.shape[1]), jnp.float32)], axis=0)


@jax.jit
def _run(x, edge_index, Wl0, bl0, Wr0, Wl1, bl1, Wr1, Wl2, bl2, Wr2,
         Wl3, bl3, Wr3):
  xp = jnp.concatenate(
      [x, jnp.zeros((_NP - _N, 128), jnp.float32)], axis=0)
  src = edge_index[0].reshape(_NWORK, _NCH, _CHUNK)
  dst = edge_index[1].reshape(_NWORK, _NCH, _CHUNK)

  wl1t = _pad_cols(Wl1.T, 64)
  bl1p = _pad_cols(bl1.reshape(1, 64), 64)
  wr1t = _pad_cols(Wr1.T, 64)
  wl2t = _pad_rows(Wl2.T, 64)
  wr2t = _pad_rows(Wr2.T, 64)

  agg128c = _make_agg(128, True)
  agg128 = _make_agg(128, False)

  p0, cnt = agg128c(xp, src, dst)
  cnt = cnt.reshape(2, _NP)
  h0, t1 = _stage_a(p0, cnt, xp, Wl0.T, bl0.reshape(1, 128), Wr0.T, wl1t)
  (p1,) = agg128(t1, src, dst)
  h1 = _stage_b(p1, cnt, h0, bl1p, wr1t)
  (p2,) = agg128(h1, src, dst)
  h2 = _stage_c(p2, cnt, h1, wl2t, bl2.reshape(1, 128), wr2t)
  (p3,) = agg128(h2, src, dst)
  out = _stage_d(p3, cnt, h2, Wl3.T, bl3.reshape(1, 128), Wr3.T)
  return out[:_N]


def kernel(x, edge_index, Wl0, bl0, Wr0, Wl1, bl1, Wr1, Wl2, bl2, Wr2,
           Wl3, bl3, Wr3):
  return _run(x, edge_index, Wl0, bl0, Wr0, Wl1, bl1, Wr1, Wl2, bl2, Wr2,
              Wl3, bl3, Wr3)


# speedup vs baseline: 12.2337x; 1.6501x over previous
import functools

import jax
import jax.numpy as jnp
from jax import lax
from jax.experimental import pallas as pl
from jax.experimental.pallas import tpu as pltpu
from jax.experimental.pallas import tpu_sc as plsc

_N = 10000
_E = 320000
_NP = 10240
_CHUNK = 80
_NWORK = 32
_EPW = _E // _NWORK
_NCH = _EPW // _CHUNK
_ROWS_PT = _NP // 16


def _make_agg(d, with_counts):
  mesh = plsc.VectorSubcoreMesh(core_axis_name="c", subcore_axis_name="s")
  out_type = [jax.ShapeDtypeStruct((2, _NP, d), jnp.float32)]
  scratch = [
      pltpu.VMEM((_EPW,), jnp.int32),
      pltpu.VMEM((_NCH, _CHUNK), jnp.int32),
      pltpu.VMEM((_CHUNK, d), jnp.float32),
      pltpu.VMEM((_CHUNK, d), jnp.float32),
      pltpu.VMEM_SHARED((_NP, d), jnp.float32),
      pltpu.SemaphoreType.DMA,
      pltpu.SemaphoreType.DMA,
  ]
  if with_counts:
    out_type.append(jax.ShapeDtypeStruct((2 * _NP,), jnp.float32))
    scratch += [
        pltpu.VMEM((_CHUNK,), jnp.float32),
        pltpu.VMEM((_ROWS_PT,), jnp.float32),
        pltpu.VMEM_SHARED((_NP,), jnp.float32),
    ]

  def body(x_hbm, src_hbm, dst_hbm, out_hbm, *rest):
    if with_counts:
      (cnt_hbm, src_v, dst_v, rows_a, rows_b, acc, sem_a, sem_b,
       ones_v, zcnt_v, cnt_acc) = rest
    else:
      src_v, dst_v, rows_a, rows_b, acc, sem_a, sem_b = rest
    cid = lax.axis_index("c")
    sid = lax.axis_index("s")
    wid = sid * 2 + cid
    tid = sid

    pltpu.sync_copy(src_hbm.at[pl.ds(wid * _EPW, _EPW)], src_v)
    pltpu.sync_copy(dst_hbm.at[wid], dst_v)

    def zrow(r, c):
      for cc in range(d // 16):
        rows_a[r, pl.ds(cc * 16, 16)] = jnp.zeros((16,), jnp.float32)
      return c
    lax.fori_loop(0, _CHUNK, zrow, 0)
    for j in range(_ROWS_PT // _CHUNK):
      pltpu.sync_copy(
          rows_a, acc.at[pl.ds(tid * _ROWS_PT + j * _CHUNK, _CHUNK)])
    if with_counts:
      def zc(i, c):
        zcnt_v[pl.ds(i * 16, 16)] = jnp.zeros((16,), jnp.float32)
        return c
      lax.fori_loop(0, _ROWS_PT // 16, zc, 0)
      pltpu.sync_copy(zcnt_v, cnt_acc.at[pl.ds(tid * _ROWS_PT, _ROWS_PT)])
      for i in range(_CHUNK // 16):
        ones_v[pl.ds(i * 16, 16)] = jnp.ones((16,), jnp.float32)

    def fire(k, buf, sem):
      pltpu.async_copy(x_hbm.at[src_v.at[pl.ds(k * _CHUNK, _CHUNK)]], buf, sem)

    def wait(buf, sem):
      pltpu.make_async_copy(x_hbm.at[pl.ds(0, _CHUNK)], buf, sem).wait()

    def drain(k, buf):
      pltpu.sync_copy(buf, acc.at[dst_v.at[k]], add=True)
      if with_counts:
        pltpu.sync_copy(ones_v, cnt_acc.at[dst_v.at[k]], add=True)

    fire(0, rows_a, sem_a)
    fire(1, rows_b, sem_b)
    plsc.subcore_barrier()

    def pair(k2, c):
      k = k2 * 2
      wait(rows_a, sem_a)
      drain(k, rows_a)
      fire(k + 2, rows_a, sem_a)
      wait(rows_b, sem_b)
      drain(k + 1, rows_b)
      @pl.when(k2 < _NCH // 2 - 1)
      def _():
        fire(k + 3, rows_b, sem_b)
      return c
    lax.fori_loop(0, _NCH // 2, pair, 0)
    wait(rows_a, sem_a)
    drain(_NCH - 1, rows_a)

    plsc.subcore_barrier()
    row0 = tid * _ROWS_PT
    pltpu.sync_copy(acc.at[pl.ds(row0, _ROWS_PT)],
                    out_hbm.at[cid, pl.ds(row0, _ROWS_PT)])
    if with_counts:
      pltpu.sync_copy(cnt_acc.at[pl.ds(row0, _ROWS_PT)],
                      cnt_hbm.at[pl.ds(cid * _NP + row0, _ROWS_PT)])

  return pl.kernel(body, out_type=tuple(out_type), mesh=mesh,
                   scratch_types=tuple(scratch))


_BN = 2048


def _inv_of(cnt_blk):
  c = cnt_blk[0] + cnt_blk[1]
  return (1.0 / jnp.maximum(c, 1.0))[:, None]


def _dot(a, b):
  return jax.lax.dot_general(a, b, (((1,), (0,)), ((), ())),
                             preferred_element_type=jnp.float32)


def _tc_call(body, out_shapes, in_specs, out_specs):
  return pl.pallas_call(
      body,
      grid=(_NP // _BN,),
      in_specs=in_specs,
      out_specs=out_specs,
      out_shape=out_shapes,
  )


def _spec_rows(d):
  return pl.BlockSpec((_BN, d), lambda i: (i, 0))


def _spec_pair(d):
  return pl.BlockSpec((2, _BN, d), lambda i: (0, i, 0))


def _spec_cnt():
  return pl.BlockSpec((2, _BN), lambda i: (0, i))


def _spec_full(r, c):
  return pl.BlockSpec((r, c), lambda i: (0, 0))


def _stage_a(p0, cnt, x, wl0t, bl0, wr0t, wl1t):
  def body(p_ref, c_ref, x_ref, wl_ref, b_ref, wr_ref, w1_ref, h0_ref, t1_ref):
    m = (p_ref[0] + p_ref[1]) * _inv_of(c_ref)
    h0 = jnp.maximum(
        _dot(m, wl_ref[...]) + b_ref[...] + _dot(x_ref[...], wr_ref[...]), 0.0)
    h0_ref[...] = h0
    t1_ref[...] = _dot(h0, w1_ref[...])
  return _tc_call(
      body,
      (jax.ShapeDtypeStruct((_NP, 128), jnp.float32),
       jax.ShapeDtypeStruct((_NP, 128), jnp.float32)),
      [_spec_pair(128), _spec_cnt(), _spec_rows(128), _spec_full(128, 128),
       _spec_full(1, 128), _spec_full(128, 128), _spec_full(128, 128)],
      (_spec_rows(128), _spec_rows(128)),
  )(p0, cnt, x, wl0t, bl0, wr0t, wl1t)


def _stage_b(p1, cnt, h0, bl1, wr1t):
  def body(p_ref, c_ref, h_ref, b_ref, wr_ref, o_ref):
    m = (p_ref[0] + p_ref[1]) * _inv_of(c_ref)
    o_ref[...] = m + b_ref[...] + _dot(h_ref[...], wr_ref[...])
  return _tc_call(
      body,
      jax.ShapeDtypeStruct((_NP, 128), jnp.float32),
      [_spec_pair(128), _spec_cnt(), _spec_rows(128), _spec_full(1, 128),
       _spec_full(128, 128)],
      _spec_rows(128),
  )(p1, cnt, h0, bl1, wr1t)


def _stage_c(p2, cnt, h1, wl2t, bl2, wr2t):
  def body(p_ref, c_ref, h_ref, wl_ref, b_ref, wr_ref, o_ref):
    m = (p_ref[0] + p_ref[1]) * _inv_of(c_ref)
    o_ref[...] = jnp.maximum(
        _dot(m, wl_ref[...]) + b_ref[...] + _dot(h_ref[...], wr_ref[...]), 0.0)
  return _tc_call(
      body,
      jax.ShapeDtypeStruct((_NP, 128), jnp.float32),
      [_spec_pair(128), _spec_cnt(), _spec_rows(128), _spec_full(128, 128),
       _spec_full(1, 128), _spec_full(128, 128)],
      _spec_rows(128),
  )(p2, cnt, h1, wl2t, bl2, wr2t)


def _stage_d(p3, cnt, h2, wl3t, bl3, wr3t):
  def body(p_ref, c_ref, h_ref, wl_ref, b_ref, wr_ref, o_ref):
    m = (p_ref[0] + p_ref[1]) * _inv_of(c_ref)
    o_ref[...] = (_dot(m, wl_ref[...]) + b_ref[...]
                  + _dot(h_ref[...], wr_ref[...]))
  return _tc_call(
      body,
      jax.ShapeDtypeStruct((_NP, 128), jnp.float32),
      [_spec_pair(128), _spec_cnt(), _spec_rows(128), _spec_full(128, 128),
       _spec_full(1, 128), _spec_full(128, 128)],
      _spec_rows(128),
  )(p3, cnt, h2, wl3t, bl3, wr3t)


def _pad_cols(w, n):
  return jnp.concatenate([w, jnp.zeros((w.shape[0], n), jnp.float32)], axis=1)


def _pad_rows(w, n):
  return jnp.concatenate([w, jnp.zeros((n, w.shape[1]), jnp.float32)], axis=0)


@jax.jit
def _run(x, edge_index, Wl0, bl0, Wr0, Wl1, bl1, Wr1, Wl2, bl2, Wr2,
         Wl3, bl3, Wr3):
  xp = jnp.concatenate(
      [x, jnp.zeros((_NP - _N, 128), jnp.float32)], axis=0)
  src = edge_index[0]
  dst = edge_index[1].reshape(_NWORK, _NCH, _CHUNK)

  wl1t = _pad_cols(Wl1.T, 64)
  bl1p = _pad_cols(bl1.reshape(1, 64), 64)
  wr1t = _pad_cols(Wr1.T, 64)
  wl2t = _pad_rows(Wl2.T, 64)
  wr2t = _pad_rows(Wr2.T, 64)

  agg128c = _make_agg(128, True)
  agg128 = _make_agg(128, False)

  p0, cnt = agg128c(xp, src, dst)
  cnt = cnt.reshape(2, _NP)
  h0, t1 = _stage_a(p0, cnt, xp, Wl0.T, bl0.reshape(1, 128), Wr0.T, wl1t)
  (p1,) = agg128(t1, src, dst)
  h1 = _stage_b(p1, cnt, h0, bl1p, wr1t)
  (p2,) = agg128(h1, src, dst)
  h2 = _stage_c(p2, cnt, h1, wl2t, bl2.reshape(1, 128), wr2t)
  (p3,) = agg128(h2, src, dst)
  out = _stage_d(p3, cnt, h2, Wl3.T, bl3.reshape(1, 128), Wr3.T)
  return out[:_N]


def kernel(x, edge_index, Wl0, bl0, Wr0, Wl1, bl1, Wr1, Wl2, bl2, Wr2,
           Wl3, bl3, Wr3):
  return _run(x, edge_index, Wl0, bl0, Wr0, Wl1, bl1, Wr1, Wl2, bl2, Wr2,
              Wl3, bl3, Wr3)


# speedup vs baseline: 13.3151x; 1.0884x over previous
import functools

import jax
import jax.numpy as jnp
from jax import lax
from jax.experimental import pallas as pl
from jax.experimental.pallas import tpu as pltpu
from jax.experimental.pallas import tpu_sc as plsc

_N = 10000
_E = 320000
_NP = 10240
_CHUNK = 80
_NWORK = 32
_EPW = _E // _NWORK
_NCH = _EPW // _CHUNK
_ROWS_PT = _NP // 16


def _make_agg(d, with_counts):
  mesh = plsc.VectorSubcoreMesh(core_axis_name="c", subcore_axis_name="s")
  out_type = [jax.ShapeDtypeStruct((2, _NP, d), jnp.float32)]
  scratch = [
      pltpu.VMEM((_EPW,), jnp.int32),
      pltpu.VMEM((_NCH, _CHUNK), jnp.int32),
      pltpu.VMEM((_CHUNK, d), jnp.float32),
      pltpu.VMEM((_CHUNK, d), jnp.float32),
      pltpu.VMEM_SHARED((_NP, d), jnp.float32),
      pltpu.SemaphoreType.DMA,
      pltpu.SemaphoreType.DMA,
  ]
  if with_counts:
    out_type.append(jax.ShapeDtypeStruct((2 * _NP,), jnp.float32))
    scratch += [
        pltpu.VMEM((_CHUNK,), jnp.float32),
        pltpu.VMEM((_ROWS_PT,), jnp.float32),
        pltpu.VMEM_SHARED((_NP,), jnp.float32),
    ]

  def body(x_hbm, src_hbm, dst_hbm, out_hbm, *rest):
    if with_counts:
      (cnt_hbm, src_v, dst_v, rows_a, rows_b, acc, sem_a, sem_b,
       ones_v, zcnt_v, cnt_acc) = rest
    else:
      src_v, dst_v, rows_a, rows_b, acc, sem_a, sem_b = rest
    cid = lax.axis_index("c")
    sid = lax.axis_index("s")
    wid = sid * 2 + cid
    tid = sid

    pltpu.sync_copy(src_hbm.at[pl.ds(wid * _EPW, _EPW)], src_v)
    pltpu.sync_copy(dst_hbm.at[wid], dst_v)

    def zrow(r, c):
      for cc in range(d // 16):
        rows_a[r, pl.ds(cc * 16, 16)] = jnp.zeros((16,), jnp.float32)
      return c
    lax.fori_loop(0, _CHUNK, zrow, 0)
    for j in range(_ROWS_PT // _CHUNK):
      pltpu.sync_copy(
          rows_a, acc.at[pl.ds(tid * _ROWS_PT + j * _CHUNK, _CHUNK)])
    if with_counts:
      def zc(i, c):
        zcnt_v[pl.ds(i * 16, 16)] = jnp.zeros((16,), jnp.float32)
        return c
      lax.fori_loop(0, _ROWS_PT // 16, zc, 0)
      pltpu.sync_copy(zcnt_v, cnt_acc.at[pl.ds(tid * _ROWS_PT, _ROWS_PT)])
      for i in range(_CHUNK // 16):
        ones_v[pl.ds(i * 16, 16)] = jnp.ones((16,), jnp.float32)

    def fire(k, buf, sem):
      pltpu.async_copy(x_hbm.at[src_v.at[pl.ds(k * _CHUNK, _CHUNK)]], buf, sem)

    def wait(buf, sem):
      pltpu.make_async_copy(x_hbm.at[pl.ds(0, _CHUNK)], buf, sem).wait()

    def drain(k, buf):
      pltpu.sync_copy(buf, acc.at[dst_v.at[k]], add=True)
      if with_counts:
        pltpu.sync_copy(ones_v, cnt_acc.at[dst_v.at[k]], add=True)

    fire(0, rows_a, sem_a)
    fire(1, rows_b, sem_b)
    plsc.subcore_barrier()

    def pair(k2, c):
      k = k2 * 2
      wait(rows_a, sem_a)
      drain(k, rows_a)
      fire(k + 2, rows_a, sem_a)
      wait(rows_b, sem_b)
      drain(k + 1, rows_b)
      @pl.when(k2 < _NCH // 2 - 1)
      def _():
        fire(k + 3, rows_b, sem_b)
      return c
    lax.fori_loop(0, _NCH // 2, pair, 0)
    wait(rows_a, sem_a)
    drain(_NCH - 1, rows_a)

    plsc.subcore_barrier()
    row0 = tid * _ROWS_PT
    pltpu.sync_copy(acc.at[pl.ds(row0, _ROWS_PT)],
                    out_hbm.at[cid, pl.ds(row0, _ROWS_PT)])
    if with_counts:
      pltpu.sync_copy(cnt_acc.at[pl.ds(row0, _ROWS_PT)],
                      cnt_hbm.at[pl.ds(cid * _NP + row0, _ROWS_PT)])

  params = pltpu.CompilerParams(use_tc_tiling_on_sc=False) if d == 64 else None
  return pl.kernel(body, out_type=tuple(out_type), mesh=mesh,
                   scratch_types=tuple(scratch), compiler_params=params)


_BN = 2048


def _inv_of(cnt_blk):
  c = cnt_blk[0] + cnt_blk[1]
  return (1.0 / jnp.maximum(c, 1.0))[:, None]


def _dot(a, b):
  return jax.lax.dot_general(a, b, (((1,), (0,)), ((), ())),
                             preferred_element_type=jnp.float32)


def _tc_call(body, out_shapes, in_specs, out_specs):
  return pl.pallas_call(
      body,
      grid=(_NP // _BN,),
      in_specs=in_specs,
      out_specs=out_specs,
      out_shape=out_shapes,
  )


def _spec_rows(d):
  return pl.BlockSpec((_BN, d), lambda i: (i, 0))


def _spec_pair(d):
  return pl.BlockSpec((2, _BN, d), lambda i: (0, i, 0))


def _spec_cnt():
  return pl.BlockSpec((2, _BN), lambda i: (0, i))


def _spec_full(r, c):
  return pl.BlockSpec((r, c), lambda i: (0, 0))


def _stage_a(p0, cnt, x, wl0t, bl0, wr0t, wl1t):
  def body(p_ref, c_ref, x_ref, wl_ref, b_ref, wr_ref, w1_ref, h0_ref, t1_ref):
    m = (p_ref[0] + p_ref[1]) * _inv_of(c_ref)
    h0 = jnp.maximum(
        _dot(m, wl_ref[...]) + b_ref[...] + _dot(x_ref[...], wr_ref[...]), 0.0)
    h0_ref[...] = h0
    t1_ref[...] = _dot(h0, w1_ref[...])
  return _tc_call(
      body,
      (jax.ShapeDtypeStruct((_NP, 128), jnp.float32),
       jax.ShapeDtypeStruct((_NP, 64), jnp.float32)),
      [_spec_pair(128), _spec_cnt(), _spec_rows(128), _spec_full(128, 128),
       _spec_full(1, 128), _spec_full(128, 128), _spec_full(128, 64)],
      (_spec_rows(128), _spec_rows(64)),
  )(p0, cnt, x, wl0t, bl0, wr0t, wl1t)


def _stage_b(p1, cnt, h0, bl1, wr1t):
  def body(p_ref, c_ref, h_ref, b_ref, wr_ref, o_ref):
    m = (p_ref[0] + p_ref[1]) * _inv_of(c_ref)
    o_ref[...] = m + b_ref[...] + _dot(h_ref[...], wr_ref[...])
  return _tc_call(
      body,
      jax.ShapeDtypeStruct((_NP, 64), jnp.float32),
      [_spec_pair(64), _spec_cnt(), _spec_rows(128), _spec_full(1, 64),
       _spec_full(128, 64)],
      _spec_rows(64),
  )(p1, cnt, h0, bl1, wr1t)


def _stage_c(p2, cnt, h1, wl2t, bl2, wr2t):
  def body(p_ref, c_ref, h_ref, wl_ref, b_ref, wr_ref, o_ref):
    m = (p_ref[0] + p_ref[1]) * _inv_of(c_ref)
    o_ref[...] = jnp.maximum(
        _dot(m, wl_ref[...]) + b_ref[...] + _dot(h_ref[...], wr_ref[...]), 0.0)
  return _tc_call(
      body,
      jax.ShapeDtypeStruct((_NP, 128), jnp.float32),
      [_spec_pair(64), _spec_cnt(), _spec_rows(64), _spec_full(64, 128),
       _spec_full(1, 128), _spec_full(64, 128)],
      _spec_rows(128),
  )(p2, cnt, h1, wl2t, bl2, wr2t)


def _stage_d(p3, cnt, h2, wl3t, bl3, wr3t):
  def body(p_ref, c_ref, h_ref, wl_ref, b_ref, wr_ref, o_ref):
    m = (p_ref[0] + p_ref[1]) * _inv_of(c_ref)
    o_ref[...] = (_dot(m, wl_ref[...]) + b_ref[...]
                  + _dot(h_ref[...], wr_ref[...]))
  return _tc_call(
      body,
      jax.ShapeDtypeStruct((_NP, 128), jnp.float32),
      [_spec_pair(128), _spec_cnt(), _spec_rows(128), _spec_full(128, 128),
       _spec_full(1, 128), _spec_full(128, 128)],
      _spec_rows(128),
  )(p3, cnt, h2, wl3t, bl3, wr3t)


def _pad_cols(w, n):
  return jnp.concatenate([w, jnp.zeros((w.shape[0], n), jnp.float32)], axis=1)


def _pad_rows(w, n):
  return jnp.concatenate([w, jnp.zeros((n, w.shape[1]), jnp.float32)], axis=0)


@jax.jit
def _run(x, edge_index, Wl0, bl0, Wr0, Wl1, bl1, Wr1, Wl2, bl2, Wr2,
         Wl3, bl3, Wr3):
  xp = jnp.concatenate(
      [x, jnp.zeros((_NP - _N, 128), jnp.float32)], axis=0)
  src = edge_index[0]
  dst = edge_index[1].reshape(_NWORK, _NCH, _CHUNK)

  agg128c = _make_agg(128, True)
  agg64 = _make_agg(64, False)
  agg128 = _make_agg(128, False)

  p0, cnt = agg128c(xp, src, dst)
  cnt = cnt.reshape(2, _NP)
  h0, t1 = _stage_a(p0, cnt, xp, Wl0.T, bl0.reshape(1, 128), Wr0.T, Wl1.T)
  (p1,) = agg64(t1, src, dst)
  h1 = _stage_b(p1, cnt, h0, bl1.reshape(1, 64), Wr1.T)
  (p2,) = agg64(h1, src, dst)
  h2 = _stage_c(p2, cnt, h1, Wl2.T, bl2.reshape(1, 128), Wr2.T)
  (p3,) = agg128(h2, src, dst)
  out = _stage_d(p3, cnt, h2, Wl3.T, bl3.reshape(1, 128), Wr3.T)
  return out[:_N]


def kernel(x, edge_index, Wl0, bl0, Wr0, Wl1, bl1, Wr1, Wl2, bl2, Wr2,
           Wl3, bl3, Wr3):
  return _run(x, edge_index, Wl0, bl0, Wr0, Wl1, bl1, Wr1, Wl2, bl2, Wr2,
              Wl3, bl3, Wr3)
